# baseline (speedup 1.0000x reference)
import functools

import jax
import jax.numpy as jnp
from jax import lax
from jax.experimental import pallas as pl
from jax.experimental.pallas import tpu as pltpu
from jax.experimental.pallas import tpu_sc as plsc

DIM = 64
LANES = 16
NUM_CORES = 2
NUM_SUBCORES = 16
NUM_WORKERS = NUM_CORES * NUM_SUBCORES


@functools.partial(jax.jit, static_argnames=("batch",))
def _distmult_sc(ht1_vert, r1_vert, h_ids, r_ids, t_ids, batch):
    b_per_w = batch // NUM_WORKERS
    mesh = plsc.VectorSubcoreMesh(core_axis_name="c", subcore_axis_name="s")

    @functools.partial(
        pl.kernel,
        mesh=mesh,
        out_type=jax.ShapeDtypeStruct((batch,), jnp.float32),
        scratch_types=[
            pltpu.VMEM((b_per_w,), jnp.int32),
            pltpu.VMEM((b_per_w,), jnp.int32),
            pltpu.VMEM((b_per_w,), jnp.int32),
            pltpu.VMEM((b_per_w, DIM), jnp.float32),
            pltpu.VMEM((b_per_w, DIM), jnp.float32),
            pltpu.VMEM((b_per_w, DIM), jnp.float32),
            pltpu.VMEM((b_per_w,), jnp.float32),
            pltpu.SemaphoreType.DMA,
        ],
        compiler_params=pltpu.CompilerParams(
            needs_layout_passes=False, use_tc_tiling_on_sc=False
        ),
    )
    def k(ht_hbm, r_hbm, hi_hbm, ri_hbm, ti_hbm, out_hbm,
          hi_v, ri_v, ti_v, h_v, r_v, t_v, o_v, sem):
        wid = lax.axis_index("s") * NUM_CORES + lax.axis_index("c")
        base = wid * b_per_w

        pltpu.sync_copy(hi_hbm.at[pl.ds(base, b_per_w)], hi_v)
        pltpu.sync_copy(ri_hbm.at[pl.ds(base, b_per_w)], ri_v)
        pltpu.sync_copy(ti_hbm.at[pl.ds(base, b_per_w)], ti_v)

        ch = pltpu.async_copy(ht_hbm.at[hi_v], h_v, sem)
        cr = pltpu.async_copy(r_hbm.at[ri_v], r_v, sem)
        ct = pltpu.async_copy(ht_hbm.at[ti_v], t_v, sem)
        ch.wait()
        cr.wait()
        ct.wait()

        lanes = lax.iota(jnp.int32, LANES)

        def group_body(g, carry):
            rows = g * LANES + lanes

            def d_body(d, acc):
                col = jnp.full((LANES,), 0, jnp.int32) + d
                hv = plsc.load_gather(h_v, [rows, col])
                rv = plsc.load_gather(r_v, [rows, col])
                tv = plsc.load_gather(t_v, [rows, col])
                return acc + hv * rv * tv

            acc = lax.fori_loop(0, DIM, d_body, jnp.zeros((LANES,), jnp.float32))
            plsc.store_scatter(o_v, [rows], acc)
            return carry

        lax.fori_loop(0, b_per_w // LANES, group_body, 0)
        pltpu.sync_copy(o_v, out_hbm.at[pl.ds(base, b_per_w)])

    return k(ht1_vert, r1_vert, h_ids, r_ids, t_ids)


def kernel(ht1_vert, r1_vert, h_ids, r_ids, t_ids):
    batch = h_ids.shape[0]
    return _distmult_sc(ht1_vert, r1_vert, h_ids, r_ids, t_ids, batch)


# speedup vs baseline: 1.6991x; 1.6991x over previous
import functools

import jax
import jax.numpy as jnp
from jax import lax
from jax.experimental import pallas as pl
from jax.experimental.pallas import tpu as pltpu
from jax.experimental.pallas import tpu_sc as plsc

DIM = 64
TILE_ROWS = 8
LANES = 16
NUM_CORES = 2
NUM_SUBCORES = 16
NUM_WORKERS = NUM_CORES * NUM_SUBCORES
NUM_CHUNKS = 4


@functools.partial(jax.jit, static_argnames=("batch",))
def _distmult_sc(ht1_vert, r1_vert, h_ids, r_ids, t_ids, batch):
    b_per_w = batch // NUM_WORKERS
    cb = b_per_w // NUM_CHUNKS
    mesh = plsc.VectorSubcoreMesh(core_axis_name="c", subcore_axis_name="s")

    @functools.partial(
        pl.kernel,
        mesh=mesh,
        out_type=jax.ShapeDtypeStruct((batch,), jnp.float32),
        scratch_types=[
            pltpu.VMEM((b_per_w,), jnp.int32),
            pltpu.VMEM((b_per_w,), jnp.int32),
            pltpu.VMEM((b_per_w,), jnp.int32),
            pltpu.VMEM((batch,), jnp.int32),
            pltpu.VMEM((cb, DIM), jnp.float32),
            pltpu.VMEM((cb, DIM), jnp.float32),
            pltpu.VMEM((cb, DIM), jnp.float32),
            pltpu.VMEM((cb, DIM), jnp.float32),
            pltpu.VMEM((cb, DIM), jnp.float32),
            pltpu.VMEM((cb, DIM), jnp.float32),
            pltpu.VMEM((b_per_w,), jnp.float32),
            pltpu.SemaphoreType.DMA,
            pltpu.SemaphoreType.DMA,
        ],
        compiler_params=pltpu.CompilerParams(needs_layout_passes=False),
    )
    def k(ht_hbm, r_hbm, hi_hbm, ri_hbm, ti_hbm, out_hbm,
          hi_s, ri_s, ti_s, idx_v, h0, h1, r0, r1, t0, t1, o_v, sem0, sem1):
        wid = lax.axis_index("s") * NUM_CORES + lax.axis_index("c")
        base = wid * b_per_w

        for src, dst in ((hi_hbm, hi_s), (ri_hbm, ri_s), (ti_hbm, ti_s)):
            pltpu.sync_copy(src.at[pl.ds(base, b_per_w)], dst)

        ht_view = ht_hbm.reshape(ht_hbm.shape[0] // TILE_ROWS, TILE_ROWS, DIM)
        r_view = r_hbm.reshape(r_hbm.shape[0] // TILE_ROWS, TILE_ROWS, DIM)

        bufs = [(h0, r0, t0, sem0), (h1, r1, t1, sem1)]

        def issue(c):
            hvb, rvb, tvb, sem = bufs[c % 2]

            def body(g, carry):
                s = pl.ds(c * cb + g * LANES, LANES)
                ev = hi_s[s]
                qv = ri_s[s]
                uv = ti_s[s]
                handles = []
                for l in range(LANES):
                    i = g * LANES + l
                    e = ev[l]
                    handles.append(pltpu.async_copy(
                        ht_view.at[e >> 3, e & 7], hvb.at[i], sem))
                    q = qv[l]
                    handles.append(pltpu.async_copy(
                        r_view.at[q >> 3, q & 7], rvb.at[i], sem))
                    u = uv[l]
                    handles.append(pltpu.async_copy(
                        ht_view.at[u >> 3, u & 7], tvb.at[i], sem))
                for h in handles:
                    h.wait()
                return carry

            lax.fori_loop(0, cb // LANES, body, 0)

        def drain(c):
            sem = bufs[c % 2][3]
            words = 3 * cb * DIM
            while words:
                n = min(words, batch)
                pltpu.make_async_copy(
                    hi_hbm.at[pl.ds(0, n)], idx_v.at[pl.ds(0, n)], sem
                ).wait()
                words -= n

        lanes = lax.iota(jnp.int32, LANES)

        def compute(c):
            hvb, rvb, tvb, _ = bufs[c % 2]

            def g_body(g, carry):
                rows = g * LANES + lanes

                def d_body(d, acc):
                    col = (lanes + d) & (DIM - 1)
                    hv = plsc.load_gather(hvb, [rows, col])
                    rv = plsc.load_gather(rvb, [rows, col])
                    tv = plsc.load_gather(tvb, [rows, col])
                    return acc + hv * rv * tv

                acc = lax.fori_loop(
                    0, DIM, d_body, jnp.zeros((LANES,), jnp.float32)
                )
                plsc.store_scatter(o_v, [c * cb + rows], acc)
                return carry

            lax.fori_loop(0, cb // LANES, g_body, 0)

        for c in range(NUM_CHUNKS):
            issue(c)
            compute(c)

        pltpu.sync_copy(o_v, out_hbm.at[pl.ds(base, b_per_w)])

    return k(ht1_vert, r1_vert, h_ids, r_ids, t_ids)


def kernel(ht1_vert, r1_vert, h_ids, r_ids, t_ids):
    batch = h_ids.shape[0]
    return _distmult_sc(ht1_vert, r1_vert, h_ids, r_ids, t_ids, batch)


# speedup vs baseline: 1.7332x; 1.0201x over previous
import functools

import jax
import jax.numpy as jnp
from jax import lax
from jax.experimental import pallas as pl
from jax.experimental.pallas import tpu as pltpu
from jax.experimental.pallas import tpu_sc as plsc

DIM = 64
TILE_ROWS = 8
LANES = 16
NUM_CORES = 2
NUM_SUBCORES = 16
NUM_WORKERS = NUM_CORES * NUM_SUBCORES
NUM_CHUNKS = 4
R_TILES = 125
R_TILES_PER_SUB = 8


@functools.partial(jax.jit, static_argnames=("batch",))
def _distmult_sc(ht1_vert, r1_vert, h_ids, r_ids, t_ids, batch):
    b_per_w = batch // NUM_WORKERS
    cb = b_per_w // NUM_CHUNKS
    r_rows = R_TILES * TILE_ROWS
    mesh = plsc.VectorSubcoreMesh(core_axis_name="c", subcore_axis_name="s")

    @functools.partial(
        pl.kernel,
        mesh=mesh,
        out_type=jax.ShapeDtypeStruct((batch,), jnp.float32),
        scratch_types=[
            pltpu.VMEM((b_per_w,), jnp.int32),
            pltpu.VMEM((b_per_w,), jnp.int32),
            pltpu.VMEM((b_per_w,), jnp.int32),
            pltpu.VMEM((R_TILES_PER_SUB, TILE_ROWS, DIM), jnp.float32),
            pltpu.VMEM((R_TILES_PER_SUB * TILE_ROWS * DIM,), jnp.float32),
            pltpu.VMEM((r_rows * DIM,), jnp.float32),
            pltpu.VMEM_SHARED((r_rows * DIM,), jnp.float32),
            pltpu.VMEM((cb, DIM), jnp.float32),
            pltpu.VMEM((cb, DIM), jnp.float32),
            pltpu.VMEM((b_per_w,), jnp.float32),
            pltpu.SemaphoreType.DMA,
        ],
        compiler_params=pltpu.CompilerParams(needs_layout_passes=False),
    )
    def k(ht_hbm, r_hbm, hi_hbm, ri_hbm, ti_hbm, out_hbm,
          hi_s, ri_s, ti_s, r_stage, r_local, r_all, r_shared,
          h_v, t_v, o_v, sem):
        sid = lax.axis_index("s")
        wid = sid * NUM_CORES + lax.axis_index("c")
        base = wid * b_per_w

        for src, dst in ((hi_hbm, hi_s), (ri_hbm, ri_s), (ti_hbm, ti_s)):
            pltpu.sync_copy(src.at[pl.ds(base, b_per_w)], dst)

        ht_view = ht_hbm.reshape(ht_hbm.shape[0] // TILE_ROWS, TILE_ROWS, DIM)
        r_view = r_hbm.reshape(R_TILES, TILE_ROWS, DIM)

        lanes = lax.iota(jnp.int32, LANES)

        bt = jnp.minimum(sid * R_TILES_PER_SUB, R_TILES - R_TILES_PER_SUB)
        pltpu.sync_copy(r_view.at[pl.ds(bt, R_TILES_PER_SUB)], r_stage)

        def unpad_row(j, carry):
            j3 = jnp.zeros((LANES,), jnp.int32) + (j >> 3)
            j7 = jnp.zeros((LANES,), jnp.int32) + (j & 7)
            for c in range(DIM // LANES):
                v = plsc.load_gather(r_stage, [j3, j7, c * LANES + lanes])
                r_local[pl.ds(j * DIM + c * LANES, LANES)] = v
            return carry

        lax.fori_loop(0, R_TILES_PER_SUB * TILE_ROWS, unpad_row, 0)
        blk = R_TILES_PER_SUB * TILE_ROWS * DIM
        pltpu.sync_copy(r_local, r_shared.at[pl.ds(bt * TILE_ROWS * DIM, blk)])
        plsc.subcore_barrier()
        pltpu.sync_copy(r_shared, r_all)

        def issue_group(c, g):
            s = pl.ds(c * cb + g * LANES, LANES)
            ev = hi_s[s]
            uv = ti_s[s]
            handles = []
            for l in range(LANES):
                i = g * LANES + l
                e = ev[l]
                handles.append(pltpu.async_copy(
                    ht_view.at[e >> 3, e & 7], h_v.at[i], sem))
                u = uv[l]
                handles.append(pltpu.async_copy(
                    ht_view.at[u >> 3, u & 7], t_v.at[i], sem))
            return handles

        n_groups = cb // LANES

        def issue(c):
            def body(gg, carry):
                g = gg * 2
                ha = issue_group(c, g)
                hb = issue_group(c, g + 1)
                for h in ha:
                    h.wait()
                for h in hb:
                    h.wait()
                return carry

            lax.fori_loop(0, n_groups // 2, body, 0)

        def compute(c):
            def g_body(g, carry):
                rows = g * LANES + lanes
                rbase = ri_s[pl.ds(c * cb + g * LANES, LANES)] * DIM

                def d_body(d, acc):
                    col = (lanes + d) & (DIM - 1)
                    hv = plsc.load_gather(h_v, [rows, col])
                    tv = plsc.load_gather(t_v, [rows, col])
                    rv = plsc.load_gather(r_all, [rbase + col])
                    return acc + hv * rv * tv

                acc = lax.fori_loop(
                    0, DIM, d_body, jnp.zeros((LANES,), jnp.float32)
                )
                plsc.store_scatter(o_v, [c * cb + rows], acc)
                return carry

            lax.fori_loop(0, n_groups, g_body, 0)

        for c in range(NUM_CHUNKS):
            issue(c)
            compute(c)

        pltpu.sync_copy(o_v, out_hbm.at[pl.ds(base, b_per_w)])

    return k(ht1_vert, r1_vert, h_ids, r_ids, t_ids)


def kernel(ht1_vert, r1_vert, h_ids, r_ids, t_ids):
    batch = h_ids.shape[0]
    return _distmult_sc(ht1_vert, r1_vert, h_ids, r_ids, t_ids, batch)
